# Initial kernel scaffold; baseline (speedup 1.0000x reference)
import jax
import jax.numpy as jnp
from jax.experimental import pallas as pl


def kernel(x, w1, g1, b1, m1, v1, w2, g2, b2, m2, v2, w_sc, g_sc, b_sc, m_sc, v_sc):
    raise NotImplementedError("write your pallas kernel here")


# speedup vs baseline: 8.1029x; 8.1029x over previous
import functools

import jax
import jax.numpy as jnp
from jax.experimental import pallas as pl
from jax.experimental.pallas import tpu as pltpu

_EPS = 1e-5


def _fold_bn(g, b, m, v):
    s = g / jnp.sqrt(v + _EPS)
    return s, b - m * s


def _block_body(x_ref, w1_ref, w2_ref, sb_ref, o_ref, h1_ref, *, H, W, Wp, Co):
    R = H * Wp
    xm = x_ref[0]
    xcat = jnp.concatenate([xm[0:R], xm[Wp:Wp + R], xm[2 * Wp:2 * Wp + R]],
                           axis=1)
    p1 = jnp.dot(xcat, w1_ref[...], preferred_element_type=jnp.float32)
    p1 = p1.reshape(H, Wp, 4 * Co)
    acc1 = (p1[:, 0:W, 0:Co] + p1[:, 1:W + 1, Co:2 * Co]
            + p1[:, 2:W + 2, 2 * Co:3 * Co])
    sc = p1[:, 1:W + 1, 3 * Co:4 * Co]

    s1 = sb_ref[0:1].reshape(1, 1, Co)
    b1 = sb_ref[1:2].reshape(1, 1, Co)
    h1 = jnp.maximum(acc1 * s1 + b1, 0.0).astype(jnp.bfloat16)

    h1_ref[...] = jnp.zeros((H + 2, Wp, Co), jnp.bfloat16)
    h1_ref[1:H + 1, 1:W + 1, :] = h1
    h1f = h1_ref[...].reshape((H + 2) * Wp, Co)

    xcat2 = jnp.concatenate([h1f[0:R], h1f[Wp:Wp + R], h1f[2 * Wp:2 * Wp + R]],
                            axis=1)
    p2 = jnp.dot(xcat2, w2_ref[...], preferred_element_type=jnp.float32)
    p2 = p2.reshape(H, Wp, 3 * Co)
    acc2 = (p2[:, 0:W, 0:Co] + p2[:, 1:W + 1, Co:2 * Co]
            + p2[:, 2:W + 2, 2 * Co:3 * Co])

    s2 = sb_ref[2:3].reshape(1, 1, Co)
    b2 = sb_ref[3:4].reshape(1, 1, Co)
    ssc = sb_ref[4:5].reshape(1, 1, Co)
    bsc = sb_ref[5:6].reshape(1, 1, Co)
    out = acc2 * s2 + b2 + (sc * ssc + bsc)
    o_ref[0] = jnp.maximum(out, 0.0)


def kernel(x, w1, g1, b1, m1, v1, w2, g2, b2, m2, v2,
           w_sc, g_sc, b_sc, m_sc, v_sc):
    N, Cin, H, W = x.shape
    Co = w1.shape[0]
    Wp = 64
    Hp = H + 2

    xt = jnp.transpose(x, (0, 2, 3, 1))
    xp = jnp.pad(xt, ((0, 0), (1, 1), (1, Wp - W - 1), (0, 0)))
    xf = xp.astype(jnp.bfloat16).reshape(N, Hp * Wp, Cin)

    s1, bb1 = _fold_bn(g1, b1, m1, v1)
    s2, bb2 = _fold_bn(g2, b2, m2, v2)
    ssc, bbsc = _fold_bn(g_sc, b_sc, m_sc, v_sc)
    z = jnp.zeros_like(s1)
    sb = jnp.stack([s1, bb1, s2, bb2, ssc, bbsc, z, z])

    wt1 = jnp.transpose(w1, (2, 1, 3, 0)).reshape(3 * Cin, 3 * Co)
    scb = jnp.zeros((3 * Cin, Co), jnp.float32)
    scb = scb.at[Cin:2 * Cin].set(w_sc[:, :, 0, 0].T)
    w1cat = jnp.concatenate([wt1, scb], axis=1).astype(jnp.bfloat16)
    w2cat = jnp.transpose(w2, (2, 1, 3, 0)).reshape(3 * Co, 3 * Co)
    w2cat = w2cat.astype(jnp.bfloat16)

    body = functools.partial(_block_body, H=H, W=W, Wp=Wp, Co=Co)
    out = pl.pallas_call(
        body,
        out_shape=jax.ShapeDtypeStruct((N, H, W, Co), jnp.float32),
        grid=(N,),
        in_specs=[
            pl.BlockSpec((1, Hp * Wp, Cin), lambda i: (i, 0, 0)),
            pl.BlockSpec((3 * Cin, 4 * Co), lambda i: (0, 0)),
            pl.BlockSpec((3 * Co, 3 * Co), lambda i: (0, 0)),
            pl.BlockSpec((8, Co), lambda i: (0, 0)),
        ],
        out_specs=pl.BlockSpec((1, H, W, Co), lambda i: (i, 0, 0, 0)),
        scratch_shapes=[pltpu.VMEM((Hp, Wp, Co), jnp.bfloat16)],
        compiler_params=pltpu.CompilerParams(
            dimension_semantics=("parallel",)),
    )(xf, w1cat, w2cat, sb)
    return jnp.transpose(out, (0, 3, 1, 2))
